# baseline (speedup 1.0000x reference)
import functools

import jax
import jax.numpy as jnp
from jax import lax
from jax.experimental import pallas as pl
from jax.experimental.pallas import tpu as pltpu
from jax.experimental.pallas import tpu_sc as plsc

NC, NS, L = 2, 16, 16
NW = NC * NS
TAG = 1 << 30


def _make_sc_embed(B, W, S, V, D):
    BPW = B // NW
    UPB = 2
    C = W // UPB
    RG = C * S
    G = BPW * UPB
    RPG = RG // L
    HALVES = D // L
    WPV = L // S
    NFULL = RG // 128
    REM = RG % 128
    ORPB = W * D // 128
    ORPG = ORPB // UPB
    FPP = WPV * D

    mesh = plsc.VectorSubcoreMesh(core_axis_name="c", subcore_axis_name="s")

    @functools.partial(
        pl.kernel,
        out_type=jax.ShapeDtypeStruct((B, ORPB, 128), jnp.float32),
        mesh=mesh,
        scratch_types=[
            pltpu.VMEM((2, RPG, L), jnp.int32),
            pltpu.VMEM((2, RG), jnp.int32),
            pltpu.VMEM((2, RG), jnp.float32),
            pltpu.VMEM((2, RG, D), jnp.float32),
            pltpu.VMEM((2, ORPG, 128), jnp.float32),
            pltpu.SemaphoreType.DMA,
            pltpu.SemaphoreType.DMA,
            pltpu.SemaphoreType.DMA,
            pltpu.SemaphoreType.DMA,
        ],
        compiler_params=pltpu.CompilerParams(use_tc_tiling_on_sc=False),
    )
    def sc_embed(mid_hbm, table_hbm, out_hbm,
                 mid_v, idx_v, mask_v, rows_v, out_v,
                 sem_in, sem_g, sem_out0, sem_out1):
        wid = lax.axis_index("s") * NC + lax.axis_index("c")

        def coords(g):
            return wid * BPW + g // UPB, g % UPB

        def in_copy(g, b):
            return pltpu.make_async_copy(
                mid_hbm.at[pl.ds((wid * G + g) * RPG, RPG), :],
                mid_v.at[b], sem_in)

        def build(b):
            def step(k, carry):
                mid = mid_v[b, k, :]
                keep = mid < TAG
                idx_v[b, pl.ds(k * L, L)] = mid & (TAG - 1)
                mask_v[b, pl.ds(k * L, L)] = jnp.where(keep, 1.0, 0.0)
                return carry

            lax.fori_loop(0, RPG, step, 0)

        def fire_gathers(b):
            for j in range(NFULL):
                pltpu.make_async_copy(
                    table_hbm.at[idx_v.at[b, pl.ds(j * 128, 128)]],
                    rows_v.at[b, pl.ds(j * 128, 128)], sem_g).start()
            if REM:
                pltpu.make_async_copy(
                    table_hbm.at[idx_v.at[b, pl.ds(NFULL * 128, REM)]],
                    rows_v.at[b, pl.ds(NFULL * 128, REM)], sem_g).start()

        def wait_gathers(b):
            pltpu.make_async_copy(
                table_hbm.at[pl.ds(0, RG), :], rows_v.at[b], sem_g).wait()

        def out_copy(g, b):
            bi, u = coords(g)
            return pltpu.make_async_copy(
                out_v.at[b], out_hbm.at[bi, pl.ds(u * ORPG, ORPG), :],
                sem_out0 if b == 0 else sem_out1)

        def compute(b):
            def pair(p, carry):
                r0 = p * L
                mv = mask_v[b, pl.ds(r0, L)]
                for u in range(WPV):
                    for h in range(HALVES):
                        acc = jnp.zeros((L,), jnp.float32)
                        for s in range(S):
                            acc = acc + rows_v[b, r0 + u * S + s,
                                               pl.ds(h * L, L)] * mv[u * S + s]
                        out_v[b, (p * FPP) // 128,
                              pl.ds((p * FPP) % 128 + u * D + h * L, L)] = acc
                return carry

            lax.fori_loop(0, RPG, pair, 0)

        def group(g, b, wait_prev_out, next_gather, next_in):
            wait_gathers(b)
            if next_gather:
                in_copy(g + 1, 1 - b).wait()
                build(1 - b)
                fire_gathers(1 - b)
            if wait_prev_out:
                out_copy(g - 2, b).wait()
            compute(b)
            out_copy(g, b).start()
            if next_in:
                in_copy(g + 2, b).start()

        in_copy(0, 0).start()
        in_copy(0, 0).wait()
        build(0)
        fire_gathers(0)
        in_copy(1, 1).start()

        group(0, 0, False, True, True)
        group(1, 1, False, True, True)

        def pipelined(k, carry):
            group(2 * k + 0, 0, True, True, True)
            group(2 * k + 1, 1, True, True, True)
            return carry

        lax.fori_loop(1, G // 2 - 1, pipelined, 0)

        group(G - 2, 0, True, True, False)
        group(G - 1, 1, True, False, False)

        out_copy(G - 2, 0).wait()
        out_copy(G - 1, 1).wait()

    return sc_embed


def kernel(token_ids, subword_mask, table):
    B, W, S = token_ids.shape
    V, D = table.shape
    NSLICE = 2
    BH = B // NSLICE
    embed = _make_sc_embed(BH, W, S, V, D)
    outs = []
    for k in range(NSLICE):
        ids32 = token_ids[k * BH:(k + 1) * BH].astype(jnp.int32)
        mk = subword_mask[k * BH:(k + 1) * BH]
        mid = jnp.where(mk, ids32, ids32 + TAG)
        outs.append(embed(mid.reshape(BH * W * S // L, L), table))
    return jnp.concatenate(outs, axis=0).reshape(B, W, D)


# speedup vs baseline: 1.0217x; 1.0217x over previous
import functools

import jax
import jax.numpy as jnp
from jax import lax
from jax.experimental import pallas as pl
from jax.experimental.pallas import tpu as pltpu
from jax.experimental.pallas import tpu_sc as plsc

NC, NS, L = 2, 16, 16
NW = NC * NS
TAG = 1 << 30


def _make_sc_embed(B, W, S, V, D):
    BPW = B // NW
    UPB = 1
    C = W // UPB
    RG = C * S
    G = BPW * UPB
    RPG = RG // L
    HALVES = D // L
    WPV = L // S
    NFULL = RG // 128
    REM = RG % 128
    ORPB = W * D // 128
    ORPG = ORPB // UPB
    FPP = WPV * D

    mesh = plsc.VectorSubcoreMesh(core_axis_name="c", subcore_axis_name="s")

    @functools.partial(
        pl.kernel,
        out_type=jax.ShapeDtypeStruct((B, ORPB, 128), jnp.float32),
        mesh=mesh,
        scratch_types=[
            pltpu.VMEM((2, RPG, L), jnp.int32),
            pltpu.VMEM((2, RG), jnp.int32),
            pltpu.VMEM((2, RG), jnp.float32),
            pltpu.VMEM((2, RG, D), jnp.float32),
            pltpu.VMEM((2, ORPG, 128), jnp.float32),
            pltpu.SemaphoreType.DMA,
            pltpu.SemaphoreType.DMA,
            pltpu.SemaphoreType.DMA,
            pltpu.SemaphoreType.DMA,
        ],
        compiler_params=pltpu.CompilerParams(use_tc_tiling_on_sc=False),
    )
    def sc_embed(mid_hbm, table_hbm, out_hbm,
                 mid_v, idx_v, mask_v, rows_v, out_v,
                 sem_in, sem_g, sem_out0, sem_out1):
        wid = lax.axis_index("s") * NC + lax.axis_index("c")

        def coords(g):
            return wid * BPW + g // UPB, g % UPB

        def in_copy(g, b):
            return pltpu.make_async_copy(
                mid_hbm.at[pl.ds((wid * G + g) * RPG, RPG), :],
                mid_v.at[b], sem_in)

        def build(b):
            def step(k, carry):
                mid = mid_v[b, k, :]
                keep = mid < TAG
                idx_v[b, pl.ds(k * L, L)] = mid & (TAG - 1)
                mask_v[b, pl.ds(k * L, L)] = jnp.where(keep, 1.0, 0.0)
                return carry

            lax.fori_loop(0, RPG, step, 0)

        def fire_gathers(b):
            for j in range(NFULL):
                pltpu.make_async_copy(
                    table_hbm.at[idx_v.at[b, pl.ds(j * 128, 128)]],
                    rows_v.at[b, pl.ds(j * 128, 128)], sem_g).start()
            if REM:
                pltpu.make_async_copy(
                    table_hbm.at[idx_v.at[b, pl.ds(NFULL * 128, REM)]],
                    rows_v.at[b, pl.ds(NFULL * 128, REM)], sem_g).start()

        def wait_gathers(b):
            pltpu.make_async_copy(
                table_hbm.at[pl.ds(0, RG), :], rows_v.at[b], sem_g).wait()

        def out_copy(g, b):
            bi, u = coords(g)
            return pltpu.make_async_copy(
                out_v.at[b], out_hbm.at[bi, pl.ds(u * ORPG, ORPG), :],
                sem_out0 if b == 0 else sem_out1)

        def compute(b):
            def pair(p, carry):
                r0 = p * L
                mv = mask_v[b, pl.ds(r0, L)]
                for u in range(WPV):
                    for h in range(HALVES):
                        acc = jnp.zeros((L,), jnp.float32)
                        for s in range(S):
                            acc = acc + rows_v[b, r0 + u * S + s,
                                               pl.ds(h * L, L)] * mv[u * S + s]
                        out_v[b, (p * FPP) // 128,
                              pl.ds((p * FPP) % 128 + u * D + h * L, L)] = acc
                return carry

            lax.fori_loop(0, RPG, pair, 0)

        def group(g, b, wait_prev_out, next_gather, next_in):
            wait_gathers(b)
            if next_gather:
                in_copy(g + 1, 1 - b).wait()
                build(1 - b)
                fire_gathers(1 - b)
            if wait_prev_out:
                out_copy(g - 2, b).wait()
            compute(b)
            out_copy(g, b).start()
            if next_in:
                in_copy(g + 2, b).start()

        in_copy(0, 0).start()
        in_copy(0, 0).wait()
        build(0)
        fire_gathers(0)
        in_copy(1, 1).start()

        group(0, 0, False, True, True)
        group(1, 1, False, True, True)

        def pipelined(k, carry):
            group(2 * k + 0, 0, True, True, True)
            group(2 * k + 1, 1, True, True, True)
            return carry

        lax.fori_loop(1, G // 2 - 1, pipelined, 0)

        group(G - 2, 0, True, True, False)
        group(G - 1, 1, True, False, False)

        out_copy(G - 2, 0).wait()
        out_copy(G - 1, 1).wait()

    return sc_embed


def kernel(token_ids, subword_mask, table):
    B, W, S = token_ids.shape
    V, D = table.shape
    N = B * W * S
    ids32 = token_ids.astype(jnp.int32)
    mid = jnp.where(subword_mask, ids32, ids32 + TAG)
    mid2 = mid.reshape(N // L, L)
    out = _make_sc_embed(B, W, S, V, D)(mid2, table)
    return out.reshape(B, W, D)
